# baseline (speedup 1.0000x reference)
import functools

import jax
import jax.numpy as jnp
from jax import lax
from jax.experimental import pallas as pl
from jax.experimental.pallas import tpu as pltpu
from jax.experimental.pallas import tpu_sc as plsc

NU = 10000
NI = 10000
E = 320000
D = 128
H = 128
O = 64

NC = 2
NS = 16
NW = NC * NS
CHF = 64
EPW = E // NW
NIT = 160
RING = 4
QNIT = 32
CHD = 100
EPT = E // NS
DNIT = EPT // CHD
DK = 8
NP = 10240
RPT = NP // NS
RB = 1000


def _proj(x, w, b):
    def body(x_ref, w_ref, b_ref, o_ref):
        o_ref[...] = lax.dot_general(
            x_ref[...], w_ref[...], (((1,), (1,)), ((), ())),
            preferred_element_type=jnp.float32) + b_ref[...]

    return pl.pallas_call(
        body,
        grid=(NI // RB,),
        in_specs=[pl.BlockSpec((RB, D), lambda i: (i, 0)),
                  pl.BlockSpec((H, D), lambda i: (0, 0)),
                  pl.BlockSpec((1, H), lambda i: (0, 0))],
        out_specs=pl.BlockSpec((RB, H), lambda i: (i, 0)),
        out_shape=jax.ShapeDtypeStruct((NI, H), jnp.float32),
    )(x, w, b)


def _mid(acc, deg, w, b):
    def body(a_ref, d_ref, w_ref, b_ref, o_ref):
        p = a_ref[0] + a_ref[1]
        dg = d_ref[...]
        mean = p / jnp.maximum(dg, 1.0)
        h = lax.dot_general(mean, w_ref[...], (((1,), (1,)), ((), ())),
                            preferred_element_type=jnp.float32) + b_ref[...]
        h = jnp.where(dg > 0, h, 0.0)
        o_ref[...] = jnp.where(h >= 0, h, 0.01 * h)

    return pl.pallas_call(
        body,
        grid=(NU // RB,),
        in_specs=[pl.BlockSpec((2, RB, H), lambda i: (0, i, 0)),
                  pl.BlockSpec((RB, 1), lambda i: (i, 0)),
                  pl.BlockSpec((H, H), lambda i: (0, 0)),
                  pl.BlockSpec((1, H), lambda i: (0, 0))],
        out_specs=pl.BlockSpec((RB, H), lambda i: (i, 0)),
        out_shape=jax.ShapeDtypeStruct((NU, H), jnp.float32),
    )(acc, deg, w, b)


def _final(acc, deg, w1, b1, wo, bo):
    def body(a_ref, d_ref, w1_ref, b1_ref, wo_ref, bo_ref, o_ref):
        p = a_ref[0] + a_ref[1]
        dg = d_ref[...]
        mean = p / jnp.maximum(dg, 1.0)
        h = lax.dot_general(mean, w1_ref[...], (((1,), (1,)), ((), ())),
                            preferred_element_type=jnp.float32) + b1_ref[...]
        h = jnp.where(dg > 0, h, 0.0)
        o_ref[...] = lax.dot_general(h, wo_ref[...], (((1,), (1,)), ((), ())),
                                     preferred_element_type=jnp.float32) + bo_ref[...]

    return pl.pallas_call(
        body,
        grid=(NI // RB,),
        in_specs=[pl.BlockSpec((2, RB, H), lambda i: (0, i, 0)),
                  pl.BlockSpec((RB, 1), lambda i: (i, 0)),
                  pl.BlockSpec((H, H), lambda i: (0, 0)),
                  pl.BlockSpec((1, H), lambda i: (0, 0)),
                  pl.BlockSpec((O, H), lambda i: (0, 0)),
                  pl.BlockSpec((1, O), lambda i: (0, 0))],
        out_specs=pl.BlockSpec((RB, O), lambda i: (i, 0)),
        out_shape=jax.ShapeDtypeStruct((NI, O), jnp.float32),
    )(acc, deg, w1, b1, wo, bo)


def _make_deg():
    mesh = plsc.VectorSubcoreMesh(core_axis_name="c", subcore_axis_name="s")

    @functools.partial(
        pl.kernel,
        mesh=mesh,
        out_type=jax.ShapeDtypeStruct((NC, NP, H), jnp.float32),
        scratch_types=[
            pltpu.VMEM((DNIT, CHD), jnp.int32),
            pltpu.VMEM((CHD, H), jnp.float32),
            pltpu.VMEM_SHARED((NP, H), jnp.float32),
            pltpu.SemaphoreType.DMA,
        ],
    )
    def deg(dst2_hbm, ones_hbm, zero_hbm, out_hbm,
            dst_v, ones_v, acc_sh, sem):
        cid = lax.axis_index("c")
        sid = lax.axis_index("s")
        w = cid * NS + sid
        pltpu.sync_copy(zero_hbm.at[pl.ds(sid * RPT, RPT)],
                        acc_sh.at[pl.ds(sid * RPT, RPT)])
        pltpu.sync_copy(ones_hbm, ones_v)
        pltpu.sync_copy(dst2_hbm.at[pl.ds(w * DNIT, DNIT)], dst_v)
        plsc.subcore_barrier()

        def body(g2, carry):
            descs = []
            for b in range(DK):
                g = g2 * DK + b
                descs.append(pltpu.async_copy(
                    ones_v, acc_sh.at[dst_v.at[g]], sem, add=True))
            for d in descs:
                d.wait()
            return carry

        lax.fori_loop(0, DNIT // DK, body, 0)
        plsc.subcore_barrier()
        pltpu.sync_copy(acc_sh.at[pl.ds(sid * RPT, RPT)],
                        out_hbm.at[cid, pl.ds(sid * RPT, RPT)])

    return deg


def _make_segsum():
    mesh = plsc.VectorSubcoreMesh(core_axis_name="c", subcore_axis_name="s")

    @functools.partial(
        pl.kernel,
        mesh=mesh,
        out_type=jax.ShapeDtypeStruct((NC, NP, H), jnp.float32),
        scratch_types=[
            pltpu.VMEM((QNIT, CHF), jnp.int32),
            pltpu.VMEM((QNIT, CHF), jnp.int32),
            [pltpu.VMEM((CHF, H), jnp.float32)] * RING,
            pltpu.VMEM_SHARED((NP, H), jnp.float32),
            [pltpu.SemaphoreType.DMA] * RING,
            [pltpu.SemaphoreType.DMA] * RING,
        ],
    )
    def seg(table_hbm, src_hbm, dst_hbm, zero_hbm, out_hbm,
            src_v, dst_v, rows, acc_sh, gsem, ssem):
        cid = lax.axis_index("c")
        sid = lax.axis_index("s")
        wid = sid * NC + cid
        pltpu.sync_copy(zero_hbm.at[pl.ds(sid * RPT, RPT)],
                        acc_sh.at[pl.ds(sid * RPT, RPT)])
        plsc.subcore_barrier()

        def gwait(g, b):
            pltpu.make_async_copy(
                table_hbm.at[src_v.at[g]], rows[b], gsem[b]).wait()

        def swait(b):
            pltpu.make_async_copy(
                rows[b], acc_sh.at[src_v.at[0]], ssem[b]).wait()

        for h in range(NIT // QNIT):
            pltpu.sync_copy(
                src_hbm.at[pl.ds(wid * NIT + h * QNIT, QNIT)], src_v)
            pltpu.sync_copy(
                dst_hbm.at[pl.ds(wid * NIT + h * QNIT, QNIT)], dst_v)
            pltpu.async_copy(table_hbm.at[src_v.at[0]], rows[0], gsem[0])
            pltpu.async_copy(table_hbm.at[src_v.at[1]], rows[1], gsem[1])

            def body(g2, carry):
                for b in range(RING):
                    g = g2 * RING + b
                    gwait(g, b)
                    pltpu.async_copy(rows[b], acc_sh.at[dst_v.at[g]],
                                     ssem[b], add=True)
                    f = g + 2
                    bf = (b + 2) % RING

                    @pl.when(f < QNIT)
                    def _():
                        @pl.when(f >= RING)
                        def _():
                            swait(bf)
                        pltpu.async_copy(
                            table_hbm.at[src_v.at[f]], rows[bf], gsem[bf])
                return carry

            lax.fori_loop(0, QNIT // RING, body, 0)
            for b in range(RING):
                swait(b)
        plsc.subcore_barrier()
        pltpu.sync_copy(acc_sh.at[pl.ds(sid * RPT, RPT)],
                        out_hbm.at[cid, pl.ds(sid * RPT, RPT)])

    return seg


_deg_pass = _make_deg()
_segsum_u = _make_segsum()
_segsum_i = _make_segsum()


def kernel(x_user, x_item, edge_index_clicks, edge_index_clicked_by,
           Wp_user, bp_user, Wp_item, bp_item,
           W0_c, b0_c, W0_cb, b0_cb,
           W1_c, b1_c, W1_cb, b1_cb,
           W_out, b_out):
    src_c = edge_index_clicks[0].astype(jnp.int32)
    dst_c = edge_index_clicks[1].astype(jnp.int32)
    src_b = edge_index_clicked_by[0].astype(jnp.int32)
    dst_b = edge_index_clicked_by[1].astype(jnp.int32)
    zeros = jnp.zeros((NP, H), jnp.float32)
    ones = jnp.ones((CHD, H), jnp.float32)

    def shard_f(x, pad):
        x2 = x.reshape(NW, EPW)
        p = jnp.full((NW, NIT * CHF - EPW), pad, jnp.int32)
        return jnp.concatenate([x2, p], axis=1).reshape(NW * NIT, CHF)

    dst2 = jnp.concatenate([dst_b, dst_c]).reshape(NC * NS * DNIT, CHD)
    degf = _deg_pass(dst2, ones, zeros)
    deg_b = degf[0, :, 0:1]
    deg_c = degf[1, :, 0:1]

    hi = _proj(x_item, Wp_item, bp_item.reshape(1, H))
    acc_b = _segsum_u(hi, shard_f(src_b, 0), shard_f(dst_b, NP - 1), zeros)
    hu1 = _mid(acc_b, deg_b, W0_cb, b0_cb.reshape(1, H))
    acc_c = _segsum_i(hu1, shard_f(src_c, 0), shard_f(dst_c, NP - 1), zeros)
    return _final(acc_c, deg_c, W1_c, b1_c.reshape(1, H),
                  W_out, b_out.reshape(1, O))


# speedup vs baseline: 2.3090x; 2.3090x over previous
import functools

import jax
import jax.numpy as jnp
from jax import lax
from jax.experimental import pallas as pl
from jax.experimental.pallas import tpu as pltpu
from jax.experimental.pallas import tpu_sc as plsc

NU = 10000
NI = 10000
E = 320000
D = 128
H = 128
O = 64

NC = 2
NS = 16
NW = NC * NS
CHF = 80
EPW = E // NW
NIT = EPW // CHF
CHD = 100
EPT = E // NS
DNIT = EPT // CHD
DK = 8
NP = 10240
RPT = NP // NS
RB = 1000


def _proj(x, w, b):
    def body(x_ref, w_ref, b_ref, o_ref):
        o_ref[...] = lax.dot_general(
            x_ref[...], w_ref[...], (((1,), (1,)), ((), ())),
            preferred_element_type=jnp.float32) + b_ref[...]

    return pl.pallas_call(
        body,
        grid=(NI // RB,),
        in_specs=[pl.BlockSpec((RB, D), lambda i: (i, 0)),
                  pl.BlockSpec((H, D), lambda i: (0, 0)),
                  pl.BlockSpec((1, H), lambda i: (0, 0))],
        out_specs=pl.BlockSpec((RB, H), lambda i: (i, 0)),
        out_shape=jax.ShapeDtypeStruct((NI, H), jnp.float32),
    )(x, w, b)


def _mid(acc, deg, w, b):
    def body(a_ref, d_ref, w_ref, b_ref, o_ref):
        p = a_ref[0] + a_ref[1]
        dg = d_ref[...]
        mean = p / jnp.maximum(dg, 1.0)
        h = lax.dot_general(mean, w_ref[...], (((1,), (1,)), ((), ())),
                            preferred_element_type=jnp.float32) + b_ref[...]
        h = jnp.where(dg > 0, h, 0.0)
        o_ref[...] = jnp.where(h >= 0, h, 0.01 * h)

    return pl.pallas_call(
        body,
        grid=(NU // RB,),
        in_specs=[pl.BlockSpec((2, RB, H), lambda i: (0, i, 0)),
                  pl.BlockSpec((RB, 1), lambda i: (i, 0)),
                  pl.BlockSpec((H, H), lambda i: (0, 0)),
                  pl.BlockSpec((1, H), lambda i: (0, 0))],
        out_specs=pl.BlockSpec((RB, H), lambda i: (i, 0)),
        out_shape=jax.ShapeDtypeStruct((NU, H), jnp.float32),
    )(acc, deg, w, b)


def _final(acc, deg, w1, b1, wo, bo):
    def body(a_ref, d_ref, w1_ref, b1_ref, wo_ref, bo_ref, o_ref):
        p = a_ref[0] + a_ref[1]
        dg = d_ref[...]
        mean = p / jnp.maximum(dg, 1.0)
        h = lax.dot_general(mean, w1_ref[...], (((1,), (1,)), ((), ())),
                            preferred_element_type=jnp.float32) + b1_ref[...]
        h = jnp.where(dg > 0, h, 0.0)
        o_ref[...] = lax.dot_general(h, wo_ref[...], (((1,), (1,)), ((), ())),
                                     preferred_element_type=jnp.float32) + bo_ref[...]

    return pl.pallas_call(
        body,
        grid=(NI // RB,),
        in_specs=[pl.BlockSpec((2, RB, H), lambda i: (0, i, 0)),
                  pl.BlockSpec((RB, 1), lambda i: (i, 0)),
                  pl.BlockSpec((H, H), lambda i: (0, 0)),
                  pl.BlockSpec((1, H), lambda i: (0, 0)),
                  pl.BlockSpec((O, H), lambda i: (0, 0)),
                  pl.BlockSpec((1, O), lambda i: (0, 0))],
        out_specs=pl.BlockSpec((RB, O), lambda i: (i, 0)),
        out_shape=jax.ShapeDtypeStruct((NI, O), jnp.float32),
    )(acc, deg, w1, b1, wo, bo)


def _make_deg():
    mesh = plsc.VectorSubcoreMesh(core_axis_name="c", subcore_axis_name="s")

    @functools.partial(
        pl.kernel,
        mesh=mesh,
        out_type=jax.ShapeDtypeStruct((NC, NP, H), jnp.float32),
        scratch_types=[
            pltpu.VMEM((DNIT, CHD), jnp.int32),
            pltpu.VMEM((CHD, H), jnp.float32),
            pltpu.VMEM_SHARED((NP, H), jnp.float32),
            pltpu.SemaphoreType.DMA,
        ],
    )
    def deg(dst2_hbm, ones_hbm, zero_hbm, out_hbm,
            dst_v, ones_v, acc_sh, sem):
        cid = lax.axis_index("c")
        sid = lax.axis_index("s")
        w = cid * NS + sid
        pltpu.sync_copy(zero_hbm.at[pl.ds(sid * RPT, RPT)],
                        acc_sh.at[pl.ds(sid * RPT, RPT)])
        pltpu.sync_copy(ones_hbm, ones_v)
        pltpu.sync_copy(dst2_hbm.at[pl.ds(w * DNIT, DNIT)], dst_v)
        plsc.subcore_barrier()

        def body(g2, carry):
            descs = []
            for b in range(DK):
                g = g2 * DK + b
                descs.append(pltpu.async_copy(
                    ones_v, acc_sh.at[dst_v.at[g]], sem, add=True))
            for d in descs:
                d.wait()
            return carry

        lax.fori_loop(0, DNIT // DK, body, 0)
        plsc.subcore_barrier()
        pltpu.sync_copy(acc_sh.at[pl.ds(sid * RPT, RPT)],
                        out_hbm.at[cid, pl.ds(sid * RPT, RPT)])

    return deg


def _make_segsum():
    mesh = plsc.VectorSubcoreMesh(core_axis_name="c", subcore_axis_name="s")
    NQ = 4

    @functools.partial(
        pl.kernel,
        mesh=mesh,
        out_type=jax.ShapeDtypeStruct((NC, NP, H), jnp.float32),
        scratch_types=[
            [pltpu.VMEM((CHF,), jnp.int32)] * NQ,
            [pltpu.VMEM((CHF,), jnp.int32)] * NQ,
            [pltpu.VMEM((CHF, H), jnp.float32)] * 2,
            pltpu.VMEM_SHARED((NP, H), jnp.float32),
            [pltpu.SemaphoreType.DMA] * NQ,
            [pltpu.SemaphoreType.DMA] * NQ,
            [pltpu.SemaphoreType.DMA] * 2,
        ],
    )
    def seg(table_hbm, src_hbm, dst_hbm, zero_hbm, out_hbm,
            src_q, dst_q, rows, acc_sh, isems, idems, gsem):
        cid = lax.axis_index("c")
        sid = lax.axis_index("s")
        wid = sid * NC + cid
        base = wid * EPW
        pltpu.sync_copy(zero_hbm.at[pl.ds(sid * RPT, RPT)],
                        acc_sh.at[pl.ds(sid * RPT, RPT)])
        plsc.subcore_barrier()

        def fire_idx(g, q):
            pltpu.async_copy(src_hbm.at[pl.ds(base + g * CHF, CHF)],
                             src_q[q], isems[q])
            pltpu.async_copy(dst_hbm.at[pl.ds(base + g * CHF, CHF)],
                             dst_q[q], idems[q])

        def wait_src(q):
            pltpu.make_async_copy(src_hbm.at[pl.ds(0, CHF)],
                                  src_q[q], isems[q]).wait()

        def wait_dst(q):
            pltpu.make_async_copy(dst_hbm.at[pl.ds(0, CHF)],
                                  dst_q[q], idems[q]).wait()

        def fire_gather(q, b):
            pltpu.async_copy(table_hbm.at[src_q[q]], rows[b], gsem[b])

        def wait_gather(b):
            pltpu.make_async_copy(table_hbm.at[src_q[0]],
                                  rows[b], gsem[b]).wait()

        for q in range(NQ):
            fire_idx(q, q)
        for j in range(2):
            wait_src(j)
            fire_gather(j, j)

        def body(g2, carry):
            for j in range(NQ):
                g = g2 * NQ + j
                b = j % 2
                wait_gather(b)
                wait_dst(j)
                pltpu.sync_copy(rows[b], acc_sh.at[dst_q[j]], add=True)
                nx4 = g + NQ

                @pl.when(nx4 < NIT)
                def _():
                    fire_idx(nx4, j)
                nx2 = g + 2

                @pl.when(nx2 < NIT)
                def _():
                    wait_src((j + 2) % NQ)
                    fire_gather((j + 2) % NQ, b)
            return carry

        lax.fori_loop(0, NIT // NQ, body, 0)
        wait_gather(0)
        wait_dst(0)
        pltpu.sync_copy(rows[0], acc_sh.at[dst_q[0]], add=True)
        plsc.subcore_barrier()
        pltpu.sync_copy(acc_sh.at[pl.ds(sid * RPT, RPT)],
                        out_hbm.at[cid, pl.ds(sid * RPT, RPT)])

    return seg


_deg_pass = _make_deg()
_segsum_u = _make_segsum()
_segsum_i = _make_segsum()


def kernel(x_user, x_item, edge_index_clicks, edge_index_clicked_by,
           Wp_user, bp_user, Wp_item, bp_item,
           W0_c, b0_c, W0_cb, b0_cb,
           W1_c, b1_c, W1_cb, b1_cb,
           W_out, b_out):
    src_c = edge_index_clicks[0].astype(jnp.int32)
    dst_c = edge_index_clicks[1].astype(jnp.int32)
    src_b = edge_index_clicked_by[0].astype(jnp.int32)
    dst_b = edge_index_clicked_by[1].astype(jnp.int32)
    zeros = jnp.zeros((NP, H), jnp.float32)
    ones = jnp.ones((CHD, H), jnp.float32)

    dst2 = jnp.concatenate([dst_b, dst_c]).reshape(NC * NS * DNIT, CHD)
    degf = _deg_pass(dst2, ones, zeros)
    deg_b = degf[0, :, 0:1]
    deg_c = degf[1, :, 0:1]

    hi = _proj(x_item, Wp_item, bp_item.reshape(1, H))
    acc_b = _segsum_u(hi, src_b, dst_b, zeros)
    hu1 = _mid(acc_b, deg_b, W0_cb, b0_cb.reshape(1, H))
    acc_c = _segsum_i(hu1, src_c, dst_c, zeros)
    return _final(acc_c, deg_c, W1_c, b1_c.reshape(1, H),
                  W_out, b_out.reshape(1, O))
